# baseline (speedup 1.0000x reference)
import functools

import jax
import jax.numpy as jnp
from jax import lax
from jax.experimental import pallas as pl
from jax.experimental.pallas import tpu as pltpu
from jax.experimental.pallas import tpu_sc as plsc

N = 10000
E = 320000
D = 128
H = 128
C = 40
G = 128

NC, NS = 2, 16
NW = NC * NS
EPW = E // NW
KCH = 2000
CH = 128
NCHUNK = E // CH
BASE_CH = NCHUNK // NW
EXTRA = NCHUNK - BASE_CH * NW
NP = 10240
RPT = NP // NS

RT = 1024
GRID = NP // RT

_mesh = plsc.VectorSubcoreMesh(core_axis_name="c", subcore_axis_name="s")


@functools.partial(
    pl.kernel,
    out_type=jax.ShapeDtypeStruct((NW, N), jnp.float32),
    mesh=_mesh,
    scratch_types=[
        pltpu.VMEM((N,), jnp.float32),
        pltpu.VMEM((KCH,), jnp.int32),
    ],
    compiler_params=pltpu.CompilerParams(needs_layout_passes=False),
)
def _sc_degree(eflat_hbm, out_hbm, counts_v, chunk_v):
    cid = lax.axis_index("c")
    sid = lax.axis_index("s")
    wid = sid * NC + cid
    zeros16 = jnp.zeros((16,), jnp.float32)
    ones16 = jnp.ones((16,), jnp.float32)

    def zbody(i, carry):
        counts_v[pl.ds(i * 16, 16)] = zeros16
        return carry

    lax.fori_loop(0, N // 16, zbody, 0)

    base = wid * EPW

    def cbody(c, carry):
        pltpu.sync_copy(eflat_hbm.at[pl.ds(E + base + c * KCH, KCH)], chunk_v)

        def ibody(j, inner):
            idx = chunk_v[pl.ds(j * 16, 16)]
            plsc.addupdate_scatter(counts_v, [idx], ones16)
            return inner

        lax.fori_loop(0, KCH // 16, ibody, 0)
        return carry

    lax.fori_loop(0, EPW // KCH, cbody, 0)
    pltpu.sync_copy(counts_v, out_hbm.at[wid])


@functools.partial(
    pl.kernel,
    out_type=jax.ShapeDtypeStruct((NC * NP, H), jnp.float32),
    mesh=_mesh,
    scratch_types=[
        pltpu.VMEM_SHARED((NP, H), jnp.float32),
        pltpu.VMEM((2, CH, H), jnp.float32),
        pltpu.VMEM((2, CH), jnp.int32),
        pltpu.VMEM((2, CH), jnp.int32),
        pltpu.VMEM((2, CH), jnp.int32),
        pltpu.SemaphoreType.DMA,
        pltpu.SemaphoreType.DMA,
        pltpu.SemaphoreType.DMA,
        pltpu.SemaphoreType.DMA,
        pltpu.SemaphoreType.DMA,
        pltpu.SemaphoreType.DMA,
    ],
    compiler_params=pltpu.CompilerParams(needs_layout_passes=False),
)
def _sc_aggregate(eflat_hbm, p_hbm, out_hbm, acc_sh, rows_v, sidx_v,
                  didx_v, sdidx_v, sem0, sem1, isem0, isem1, ssem0, ssem1):
    cid = lax.axis_index("c")
    sid = lax.axis_index("s")
    wid = sid * NC + cid
    sems = (sem0, sem1)
    isems = (isem0, isem1)
    ssems = (ssem0, ssem1)
    zeros16 = jnp.zeros((16,), jnp.float32)

    def zb(r, carry):
        def zc(k, inner):
            rows_v[0, r, pl.ds(k * 16, 16)] = zeros16
            return inner

        lax.fori_loop(0, H // 16, zc, 0)
        return carry

    lax.fori_loop(0, CH, zb, 0)

    row0 = sid * RPT
    for k in range(RPT // CH):
        pltpu.sync_copy(rows_v.at[0],
                        acc_sh.at[pl.ds(row0 + k * CH, CH)])

    ch0 = wid * BASE_CH

    def _load_idx_async(chunk, b):
        base = pl.multiple_of(chunk * CH, 8)
        pltpu.async_copy(eflat_hbm.at[pl.ds(base, CH)], sidx_v.at[b], isems[b])
        pltpu.async_copy(eflat_hbm.at[pl.ds(E + base, CH)], didx_v.at[b],
                         isems[b])

    def _wait_idx(b):
        pltpu.make_async_copy(eflat_hbm.at[pl.ds(0, CH)], sidx_v.at[b],
                              isems[b]).wait()
        pltpu.make_async_copy(eflat_hbm.at[pl.ds(0, CH)], didx_v.at[b],
                              isems[b]).wait()

    def _fire(b):
        pltpu.async_copy(p_hbm.at[sidx_v.at[b]], rows_v.at[b], sems[b])

    def _drain_gather(b):
        pltpu.make_async_copy(p_hbm.at[sidx_v.at[b]], rows_v.at[b],
                              sems[b]).wait()

    def _fire_scatter(b):
        for j in range(CH // 16):
            sdidx_v[b, pl.ds(j * 16, 16)] = didx_v[b, pl.ds(j * 16, 16)]
        pltpu.async_copy(rows_v.at[b], acc_sh.at[sdidx_v.at[b]], ssems[b],
                         add=True)

    def _wait_scatter(b):
        pltpu.make_async_copy(rows_v.at[b], acc_sh.at[sdidx_v.at[b]],
                              ssems[b]).wait()

    _load_idx_async(ch0, 0)
    _load_idx_async(ch0 + 1, 1)
    _wait_idx(0)
    _fire(0)
    _wait_idx(1)
    _fire(1)
    plsc.subcore_barrier()

    def cbody(s, carry):
        last = BASE_CH // 2 - 1
        _drain_gather(0)
        _fire_scatter(0)

        @pl.when(s < last)
        def _():
            _load_idx_async(ch0 + 2 * s + 2, 0)

        _drain_gather(1)
        _fire_scatter(1)

        @pl.when(s < last)
        def _():
            _load_idx_async(ch0 + 2 * s + 3, 1)
            _wait_scatter(0)
            _wait_idx(0)
            _fire(0)
            _wait_scatter(1)
            _wait_idx(1)
            _fire(1)

        return carry

    lax.fori_loop(0, BASE_CH // 2, cbody, 0)
    _wait_scatter(0)
    _wait_scatter(1)

    @pl.when(wid < EXTRA)
    def _():
        _load_idx_async(NCHUNK - EXTRA + wid, 0)
        _wait_idx(0)
        _fire(0)
        _drain_gather(0)
        _fire_scatter(0)
        _wait_scatter(0)

    plsc.subcore_barrier()

    outbase = cid * NP + row0
    pltpu.sync_copy(acc_sh.at[pl.ds(row0, RPT)],
                    out_hbm.at[pl.ds(outbase, RPT)])


@functools.partial(
    pl.kernel,
    out_type=jax.ShapeDtypeStruct((NC * NP * G,), jnp.float32),
    mesh=_mesh,
    scratch_types=[
        pltpu.VMEM_SHARED((NP * G,), jnp.float32),
        pltpu.VMEM((N,), jnp.int32),
        pltpu.VMEM((N,), jnp.float32),
        pltpu.VMEM((8192,), jnp.float32),
        pltpu.VMEM((CH,), jnp.int32),
        pltpu.VMEM((CH,), jnp.int32),
        pltpu.VMEM((2, CH), jnp.int32),
        pltpu.VMEM((2, CH), jnp.float32),
        pltpu.SemaphoreType.DMA,
        pltpu.SemaphoreType.DMA,
    ],
    compiler_params=pltpu.CompilerParams(needs_layout_passes=False),
)
def _sc_poolmat(eflat_hbm, batch_hbm, dinv_hbm, out_hbm, vacc_sh,
                batch_v, dinv_v, zbuf_v, sidx_v, didx_v, fidx_v, fval_v,
                ssem0, ssem1):
    cid = lax.axis_index("c")
    sid = lax.axis_index("s")
    wid = sid * NC + cid
    zeros16 = jnp.zeros((16,), jnp.float32)

    def zb(i, carry):
        zbuf_v[pl.ds(i * 16, 16)] = zeros16
        return carry

    lax.fori_loop(0, 8192 // 16, zb, 0)

    wpt = NP * G // NS
    off0 = sid * wpt
    for k in range(wpt // 8192):
        pltpu.sync_copy(zbuf_v, vacc_sh.at[pl.ds(off0 + k * 8192, 8192)])

    pltpu.sync_copy(batch_hbm, batch_v)
    pltpu.sync_copy(dinv_hbm.at[pl.ds(0, N)], dinv_v)
    plsc.subcore_barrier()

    ch0 = wid * BASE_CH
    ssems = (ssem0, ssem1)

    def _compute_chunk(chunk, b):
        base = pl.multiple_of(chunk * CH, 8)
        pltpu.sync_copy(eflat_hbm.at[pl.ds(base, CH)], sidx_v)
        pltpu.sync_copy(eflat_hbm.at[pl.ds(E + base, CH)], didx_v)

        def jbody(j, inner):
            d16 = didx_v[pl.ds(j * 16, 16)]
            s16 = sidx_v[pl.ds(j * 16, 16)]
            b16 = plsc.load_gather(batch_v, [d16])
            w16 = plsc.load_gather(dinv_v, [d16])
            fidx_v[b, pl.ds(j * 16, 16)] = s16 * G + b16
            fval_v[b, pl.ds(j * 16, 16)] = w16
            return inner

        lax.fori_loop(0, CH // 16, jbody, 0)

    def _fire_scatter(b):
        pltpu.async_copy(fval_v.at[b], vacc_sh.at[fidx_v.at[b]], ssems[b],
                         add=True)

    def _wait_scatter(b):
        pltpu.make_async_copy(fval_v.at[b], vacc_sh.at[fidx_v.at[b]],
                              ssems[b]).wait()

    def cbody(s, carry):
        for b in range(2):
            @pl.when(s > 0)
            def _():
                _wait_scatter(b)

            _compute_chunk(ch0 + 2 * s + b, b)
            _fire_scatter(b)
        return carry

    lax.fori_loop(0, BASE_CH // 2, cbody, 0)
    _wait_scatter(0)
    _wait_scatter(1)

    @pl.when(wid < EXTRA)
    def _():
        _compute_chunk(NCHUNK - EXTRA + wid, 0)
        _fire_scatter(0)
        _wait_scatter(0)

    plsc.subcore_barrier()
    pltpu.sync_copy(vacc_sh.at[pl.ds(off0, wpt)],
                    out_hbm.at[pl.ds(cid * NP * G + off0, wpt)])


def _tc_prep_body(counts_ref, x_ref, w1_ref, p_ref, dinv_ref):
    deg = jnp.sum(counts_ref[...], axis=1, keepdims=True) + 1.0
    dinv = lax.rsqrt(deg)
    xw = jnp.dot(x_ref[...], w1_ref[...],
                 preferred_element_type=jnp.float32)
    p_ref[...] = dinv * xw
    dinv_ref[...] = dinv


_tc_prep = pl.pallas_call(
    _tc_prep_body,
    grid=(GRID,),
    in_specs=[
        pl.BlockSpec((RT, NW), lambda i: (i, 0)),
        pl.BlockSpec((RT, D), lambda i: (i, 0)),
        pl.BlockSpec((D, H), lambda i: (0, 0)),
    ],
    out_specs=[
        pl.BlockSpec((RT, H), lambda i: (i, 0)),
        pl.BlockSpec((RT, 1), lambda i: (i, 0)),
    ],
    out_shape=[
        jax.ShapeDtypeStruct((NP, H), jnp.float32),
        jax.ShapeDtypeStruct((NP, 1), jnp.float32),
    ],
)


def _tc_pool_body(a0_ref, a1_ref, p_ref, v0_ref, v1_ref, dinv_ref, batch_ref,
                  b1_ref, w2_ref, b2_ref, w3_ref, b3_ref, out_ref, sums_ref,
                  cnts_ref):
    i = pl.program_id(0)

    @pl.when(i == 0)
    def _():
        sums_ref[...] = jnp.zeros_like(sums_ref)
        cnts_ref[...] = jnp.zeros_like(cnts_ref)

    dinv = dinv_ref[...]
    h1 = dinv * (a0_ref[...] + a1_ref[...] + p_ref[...]) + b1_ref[0, :][None, :]
    q = dinv * jnp.maximum(h1, 0.0)
    gids = lax.broadcasted_iota(jnp.int32, (RT, G), 1)
    sel = jnp.where(batch_ref[...] == gids, 1.0, 0.0)
    vt = v0_ref[...] + v1_ref[...] + sel * dinv
    sums_ref[...] += lax.dot_general(vt, q, (((0,), (0,)), ((), ())),
                                     preferred_element_type=jnp.float32)
    cnts_ref[...] += jnp.sum(sel, axis=0)[None, :]

    @pl.when(i == GRID - 1)
    def _():
        cnt = cnts_ref[0, :]
        pooled = sums_ref[...] / jnp.maximum(cnt, 1.0)[:, None]
        w23 = jnp.dot(w2_ref[...], w3_ref[...],
                      preferred_element_type=jnp.float32)
        bb = jnp.dot(b2_ref[...], w3_ref[...],
                     preferred_element_type=jnp.float32)
        nonempty = jnp.where(cnt > 0.0, 1.0, 0.0)
        out_ref[...] = (jnp.dot(pooled, w23,
                                preferred_element_type=jnp.float32)
                        + nonempty[:, None] * bb + b3_ref[...])


_tc_pool = pl.pallas_call(
    _tc_pool_body,
    grid=(GRID,),
    in_specs=[
        pl.BlockSpec((RT, H), lambda i: (i, 0)),
        pl.BlockSpec((RT, H), lambda i: (i + NP // RT, 0)),
        pl.BlockSpec((RT, H), lambda i: (i, 0)),
        pl.BlockSpec((RT, G), lambda i: (i, 0)),
        pl.BlockSpec((RT, G), lambda i: (i + NP // RT, 0)),
        pl.BlockSpec((RT, 1), lambda i: (i, 0)),
        pl.BlockSpec((RT, 1), lambda i: (i, 0)),
        pl.BlockSpec((1, H), lambda i: (0, 0)),
        pl.BlockSpec((H, H), lambda i: (0, 0)),
        pl.BlockSpec((1, H), lambda i: (0, 0)),
        pl.BlockSpec((H, C), lambda i: (0, 0)),
        pl.BlockSpec((1, C), lambda i: (0, 0)),
    ],
    out_specs=pl.BlockSpec((G, C), lambda i: (0, 0)),
    out_shape=jax.ShapeDtypeStruct((G, C), jnp.float32),
    scratch_shapes=[
        pltpu.VMEM((G, H), jnp.float32),
        pltpu.VMEM((1, G), jnp.float32),
    ],
    compiler_params=pltpu.CompilerParams(fuse_transposed_lhs_in_matmul=True),
)


def kernel(x, edge_index, batch, W1, b1, W2, b2, W3, b3):
    eflat = edge_index.reshape(2 * E)
    xp = jnp.pad(x, ((0, NP - N), (0, 0)))
    batchp = jnp.pad(batch, (0, NP - N), constant_values=G)
    counts = _sc_degree(eflat)
    counts_t = jnp.pad(counts.T, ((0, NP - N), (0, 0)))
    p, dinv = _tc_prep(counts_t, xp, W1)
    acc1 = _sc_aggregate(eflat, p)
    vflat = _sc_poolmat(eflat, batch, dinv.reshape(NP))
    v2d = vflat.reshape(NC * NP, G)
    out = _tc_pool(acc1, acc1, p, v2d, v2d, dinv, batchp.reshape(NP, 1),
                   b1.reshape(1, H), W2, b2.reshape(1, H), W3,
                   b3.reshape(1, C))
    return out


# speedup vs baseline: 1.1124x; 1.1124x over previous
import functools

import jax
import jax.numpy as jnp
from jax import lax
from jax.experimental import pallas as pl
from jax.experimental.pallas import tpu as pltpu
from jax.experimental.pallas import tpu_sc as plsc

N = 10000
E = 320000
D = 128
H = 128
C = 40
G = 128

NC, NS = 2, 16
NW = NC * NS
EPW = E // NW
KCH = 2000
CH = 128
NCHUNK = E // CH
BASE_CH = NCHUNK // NW
EXTRA = NCHUNK - BASE_CH * NW
NP = 10240
RPT = NP // NS

RT = 1024
GRID = NP // RT

_mesh = plsc.VectorSubcoreMesh(core_axis_name="c", subcore_axis_name="s")


@functools.partial(
    pl.kernel,
    out_type=jax.ShapeDtypeStruct((NW, N), jnp.float32),
    mesh=_mesh,
    scratch_types=[
        pltpu.VMEM((N,), jnp.float32),
        pltpu.VMEM((KCH,), jnp.int32),
    ],
    compiler_params=pltpu.CompilerParams(needs_layout_passes=False),
)
def _sc_degree(eflat_hbm, out_hbm, counts_v, chunk_v):
    cid = lax.axis_index("c")
    sid = lax.axis_index("s")
    wid = sid * NC + cid
    zeros16 = jnp.zeros((16,), jnp.float32)
    ones16 = jnp.ones((16,), jnp.float32)

    def zbody(i, carry):
        counts_v[pl.ds(i * 16, 16)] = zeros16
        return carry

    lax.fori_loop(0, N // 16, zbody, 0)

    base = wid * EPW

    def cbody(c, carry):
        pltpu.sync_copy(eflat_hbm.at[pl.ds(E + base + c * KCH, KCH)], chunk_v)

        def ibody(j, inner):
            idx = chunk_v[pl.ds(j * 16, 16)]
            plsc.addupdate_scatter(counts_v, [idx], ones16)
            return inner

        lax.fori_loop(0, KCH // 16, ibody, 0)
        return carry

    lax.fori_loop(0, EPW // KCH, cbody, 0)
    pltpu.sync_copy(counts_v, out_hbm.at[wid])


@functools.partial(
    pl.kernel,
    out_type=jax.ShapeDtypeStruct((NC * NP, H), jnp.float32),
    mesh=_mesh,
    scratch_types=[
        pltpu.VMEM_SHARED((NP, H), jnp.float32),
        pltpu.VMEM((2, CH, H), jnp.float32),
        pltpu.VMEM((2, CH), jnp.int32),
        pltpu.VMEM((2, CH), jnp.int32),
        pltpu.SemaphoreType.DMA,
        pltpu.SemaphoreType.DMA,
        pltpu.SemaphoreType.DMA,
        pltpu.SemaphoreType.DMA,
    ],
    compiler_params=pltpu.CompilerParams(needs_layout_passes=False),
)
def _sc_aggregate(eflat_hbm, p_hbm, out_hbm, acc_sh, rows_v, sidx_v,
                  didx_v, sem0, sem1, isem0, isem1):
    cid = lax.axis_index("c")
    sid = lax.axis_index("s")
    wid = sid * NC + cid
    sems = (sem0, sem1)
    isems = (isem0, isem1)
    zeros16 = jnp.zeros((16,), jnp.float32)

    def zb(r, carry):
        def zc(k, inner):
            rows_v[0, r, pl.ds(k * 16, 16)] = zeros16
            return inner

        lax.fori_loop(0, H // 16, zc, 0)
        return carry

    lax.fori_loop(0, CH, zb, 0)

    row0 = sid * RPT
    for k in range(RPT // CH):
        pltpu.sync_copy(rows_v.at[0],
                        acc_sh.at[pl.ds(row0 + k * CH, CH)])

    ch0 = wid * BASE_CH

    def _load_idx_async(chunk, b):
        base = pl.multiple_of(chunk * CH, 8)
        pltpu.async_copy(eflat_hbm.at[pl.ds(base, CH)], sidx_v.at[b], isems[b])
        pltpu.async_copy(eflat_hbm.at[pl.ds(E + base, CH)], didx_v.at[b],
                         isems[b])

    def _wait_idx(b):
        pltpu.make_async_copy(eflat_hbm.at[pl.ds(0, CH)], sidx_v.at[b],
                              isems[b]).wait()
        pltpu.make_async_copy(eflat_hbm.at[pl.ds(0, CH)], didx_v.at[b],
                              isems[b]).wait()

    def _fire(b):
        pltpu.async_copy(p_hbm.at[sidx_v.at[b]], rows_v.at[b], sems[b])

    def _drain_gather(b):
        pltpu.make_async_copy(p_hbm.at[sidx_v.at[b]], rows_v.at[b],
                              sems[b]).wait()

    def _scatter(b):
        pltpu.sync_copy(rows_v.at[b], acc_sh.at[didx_v.at[b]], add=True)

    _load_idx_async(ch0, 0)
    _load_idx_async(ch0 + 1, 1)
    _wait_idx(0)
    _fire(0)
    _wait_idx(1)
    _fire(1)
    plsc.subcore_barrier()

    def cbody(s, carry):
        last = BASE_CH // 2 - 1
        _drain_gather(0)

        @pl.when(s < last)
        def _():
            _load_idx_async(ch0 + 2 * s + 2, 0)

        _scatter(0)
        _drain_gather(1)

        @pl.when(s < last)
        def _():
            _load_idx_async(ch0 + 2 * s + 3, 1)
            _wait_idx(0)
            _fire(0)

        _scatter(1)

        @pl.when(s < last)
        def _():
            _wait_idx(1)
            _fire(1)

        return carry

    lax.fori_loop(0, BASE_CH // 2, cbody, 0)

    @pl.when(wid < EXTRA)
    def _():
        _load_idx_async(NCHUNK - EXTRA + wid, 0)
        _wait_idx(0)
        _fire(0)
        _drain_gather(0)
        _scatter(0)

    plsc.subcore_barrier()

    outbase = cid * NP + row0
    pltpu.sync_copy(acc_sh.at[pl.ds(row0, RPT)],
                    out_hbm.at[pl.ds(outbase, RPT)])


@functools.partial(
    pl.kernel,
    out_type=jax.ShapeDtypeStruct((NC * NP * G,), jnp.float32),
    mesh=_mesh,
    scratch_types=[
        pltpu.VMEM_SHARED((NP * G,), jnp.float32),
        pltpu.VMEM((N,), jnp.int32),
        pltpu.VMEM((N,), jnp.float32),
        pltpu.VMEM((8192,), jnp.float32),
        pltpu.VMEM((CH,), jnp.int32),
        pltpu.VMEM((CH,), jnp.int32),
        pltpu.VMEM((2, CH), jnp.int32),
        pltpu.VMEM((2, CH), jnp.float32),
        pltpu.SemaphoreType.DMA,
        pltpu.SemaphoreType.DMA,
    ],
    compiler_params=pltpu.CompilerParams(needs_layout_passes=False),
)
def _sc_poolmat(eflat_hbm, batch_hbm, dinv_hbm, out_hbm, vacc_sh,
                batch_v, dinv_v, zbuf_v, sidx_v, didx_v, fidx_v, fval_v,
                ssem0, ssem1):
    cid = lax.axis_index("c")
    sid = lax.axis_index("s")
    wid = sid * NC + cid
    zeros16 = jnp.zeros((16,), jnp.float32)

    def zb(i, carry):
        zbuf_v[pl.ds(i * 16, 16)] = zeros16
        return carry

    lax.fori_loop(0, 8192 // 16, zb, 0)

    wpt = NP * G // NS
    off0 = sid * wpt
    for k in range(wpt // 8192):
        pltpu.sync_copy(zbuf_v, vacc_sh.at[pl.ds(off0 + k * 8192, 8192)])

    pltpu.sync_copy(batch_hbm, batch_v)
    pltpu.sync_copy(dinv_hbm.at[pl.ds(0, N)], dinv_v)
    plsc.subcore_barrier()

    ch0 = wid * BASE_CH
    ssems = (ssem0, ssem1)

    def _compute_chunk(chunk, b):
        base = pl.multiple_of(chunk * CH, 8)
        pltpu.sync_copy(eflat_hbm.at[pl.ds(base, CH)], sidx_v)
        pltpu.sync_copy(eflat_hbm.at[pl.ds(E + base, CH)], didx_v)

        def jbody(j, inner):
            d16 = didx_v[pl.ds(j * 16, 16)]
            s16 = sidx_v[pl.ds(j * 16, 16)]
            b16 = plsc.load_gather(batch_v, [d16])
            w16 = plsc.load_gather(dinv_v, [d16])
            fidx_v[b, pl.ds(j * 16, 16)] = s16 * G + b16
            fval_v[b, pl.ds(j * 16, 16)] = w16
            return inner

        lax.fori_loop(0, CH // 16, jbody, 0)

    def _fire_scatter(b):
        pltpu.async_copy(fval_v.at[b], vacc_sh.at[fidx_v.at[b]], ssems[b],
                         add=True)

    def _wait_scatter(b):
        pltpu.make_async_copy(fval_v.at[b], vacc_sh.at[fidx_v.at[b]],
                              ssems[b]).wait()

    def cbody(s, carry):
        for b in range(2):
            @pl.when(s > 0)
            def _():
                _wait_scatter(b)

            _compute_chunk(ch0 + 2 * s + b, b)
            _fire_scatter(b)
        return carry

    lax.fori_loop(0, BASE_CH // 2, cbody, 0)
    _wait_scatter(0)
    _wait_scatter(1)

    @pl.when(wid < EXTRA)
    def _():
        _compute_chunk(NCHUNK - EXTRA + wid, 0)
        _fire_scatter(0)
        _wait_scatter(0)

    plsc.subcore_barrier()
    pltpu.sync_copy(vacc_sh.at[pl.ds(off0, wpt)],
                    out_hbm.at[pl.ds(cid * NP * G + off0, wpt)])


def _tc_prep_body(counts_ref, x_ref, w1_ref, p_ref, dinv_ref):
    deg = jnp.sum(counts_ref[...], axis=1, keepdims=True) + 1.0
    dinv = lax.rsqrt(deg)
    xw = jnp.dot(x_ref[...], w1_ref[...],
                 preferred_element_type=jnp.float32)
    p_ref[...] = dinv * xw
    dinv_ref[...] = dinv


_tc_prep = pl.pallas_call(
    _tc_prep_body,
    grid=(GRID,),
    in_specs=[
        pl.BlockSpec((RT, NW), lambda i: (i, 0)),
        pl.BlockSpec((RT, D), lambda i: (i, 0)),
        pl.BlockSpec((D, H), lambda i: (0, 0)),
    ],
    out_specs=[
        pl.BlockSpec((RT, H), lambda i: (i, 0)),
        pl.BlockSpec((RT, 1), lambda i: (i, 0)),
    ],
    out_shape=[
        jax.ShapeDtypeStruct((NP, H), jnp.float32),
        jax.ShapeDtypeStruct((NP, 1), jnp.float32),
    ],
)


def _tc_pool_body(a0_ref, a1_ref, p_ref, v0_ref, v1_ref, dinv_ref, batch_ref,
                  b1_ref, w2_ref, b2_ref, w3_ref, b3_ref, out_ref, sums_ref,
                  cnts_ref):
    i = pl.program_id(0)

    @pl.when(i == 0)
    def _():
        sums_ref[...] = jnp.zeros_like(sums_ref)
        cnts_ref[...] = jnp.zeros_like(cnts_ref)

    dinv = dinv_ref[...]
    h1 = dinv * (a0_ref[...] + a1_ref[...] + p_ref[...]) + b1_ref[0, :][None, :]
    q = dinv * jnp.maximum(h1, 0.0)
    gids = lax.broadcasted_iota(jnp.int32, (RT, G), 1)
    sel = jnp.where(batch_ref[...] == gids, 1.0, 0.0)
    vt = v0_ref[...] + v1_ref[...] + sel * dinv
    sums_ref[...] += lax.dot_general(vt, q, (((0,), (0,)), ((), ())),
                                     preferred_element_type=jnp.float32)
    cnts_ref[...] += jnp.sum(sel, axis=0)[None, :]

    @pl.when(i == GRID - 1)
    def _():
        cnt = cnts_ref[0, :]
        pooled = sums_ref[...] / jnp.maximum(cnt, 1.0)[:, None]
        w23 = jnp.dot(w2_ref[...], w3_ref[...],
                      preferred_element_type=jnp.float32)
        bb = jnp.dot(b2_ref[...], w3_ref[...],
                     preferred_element_type=jnp.float32)
        nonempty = jnp.where(cnt > 0.0, 1.0, 0.0)
        out_ref[...] = (jnp.dot(pooled, w23,
                                preferred_element_type=jnp.float32)
                        + nonempty[:, None] * bb + b3_ref[...])


_tc_pool = pl.pallas_call(
    _tc_pool_body,
    grid=(GRID,),
    in_specs=[
        pl.BlockSpec((RT, H), lambda i: (i, 0)),
        pl.BlockSpec((RT, H), lambda i: (i + NP // RT, 0)),
        pl.BlockSpec((RT, H), lambda i: (i, 0)),
        pl.BlockSpec((RT, G), lambda i: (i, 0)),
        pl.BlockSpec((RT, G), lambda i: (i + NP // RT, 0)),
        pl.BlockSpec((RT, 1), lambda i: (i, 0)),
        pl.BlockSpec((RT, 1), lambda i: (i, 0)),
        pl.BlockSpec((1, H), lambda i: (0, 0)),
        pl.BlockSpec((H, H), lambda i: (0, 0)),
        pl.BlockSpec((1, H), lambda i: (0, 0)),
        pl.BlockSpec((H, C), lambda i: (0, 0)),
        pl.BlockSpec((1, C), lambda i: (0, 0)),
    ],
    out_specs=pl.BlockSpec((G, C), lambda i: (0, 0)),
    out_shape=jax.ShapeDtypeStruct((G, C), jnp.float32),
    scratch_shapes=[
        pltpu.VMEM((G, H), jnp.float32),
        pltpu.VMEM((1, G), jnp.float32),
    ],
    compiler_params=pltpu.CompilerParams(fuse_transposed_lhs_in_matmul=True),
)


def kernel(x, edge_index, batch, W1, b1, W2, b2, W3, b3):
    eflat = edge_index.reshape(2 * E)
    xp = jnp.pad(x, ((0, NP - N), (0, 0)))
    batchp = jnp.pad(batch, (0, NP - N), constant_values=G)
    counts = _sc_degree(eflat)
    counts_t = jnp.pad(counts.T, ((0, NP - N), (0, 0)))
    p, dinv = _tc_prep(counts_t, xp, W1)
    acc1 = _sc_aggregate(eflat, p)
    vflat = _sc_poolmat(eflat, batch, dinv.reshape(NP))
    v2d = vflat.reshape(NC * NP, G)
    out = _tc_pool(acc1, acc1, p, v2d, v2d, dinv, batchp.reshape(NP, 1),
                   b1.reshape(1, H), W2, b2.reshape(1, H), W3,
                   b3.reshape(1, C))
    return out
